# Initial kernel scaffold; baseline (speedup 1.0000x reference)
import functools

import jax
import jax.numpy as jnp
from jax import lax
from jax.experimental import pallas as pl
from jax.experimental.pallas import tpu as pltpu
from jax.experimental.pallas import tpu_sc as plsc

N_NODES = 10000
D = 128
EPS = 1e-5

NC = 2
NS = 16
NW = NC * NS

E_CHUNK = 80
ZROWS = 125


def _sc_segment_sum(x, src3, dst3, n_chunks):
    rows_per_tile = N_NODES // NS

    mesh = plsc.VectorSubcoreMesh(core_axis_name="c", subcore_axis_name="s")

    @functools.partial(
        pl.kernel,
        out_type=jax.ShapeDtypeStruct((NC, N_NODES, D), jnp.float32),
        mesh=mesh,
        scratch_types=[
            pltpu.VMEM((n_chunks, E_CHUNK), jnp.int32),
            pltpu.VMEM((n_chunks, E_CHUNK), jnp.int32),
            pltpu.VMEM((E_CHUNK, D), jnp.float32),
            pltpu.VMEM((ZROWS, D), jnp.float32),
            pltpu.VMEM_SHARED((N_NODES, D), jnp.float32),
            pltpu.SemaphoreType.DMA,
        ],
    )
    def k(x_hbm, src_hbm, dst_hbm, out_hbm, sidx, didx, rows, zbuf, agg, sem):
        cid = lax.axis_index("c")
        sid = lax.axis_index("s")
        wid = cid * NS + sid

        zero16 = jnp.zeros((16,), jnp.float32)

        def zfill(r, carry):
            for j in range(D // 16):
                zbuf[r, pl.ds(j * 16, 16)] = zero16
            return carry

        lax.fori_loop(0, ZROWS, zfill, 0)
        row0 = sid * rows_per_tile
        for j in range(rows_per_tile // ZROWS):
            pltpu.sync_copy(zbuf, agg.at[pl.ds(row0 + j * ZROWS, ZROWS)])

        pltpu.sync_copy(src_hbm.at[wid], sidx)
        pltpu.sync_copy(dst_hbm.at[wid], didx)

        plsc.subcore_barrier()

        def body(i, carry):
            pltpu.async_copy(x_hbm.at[sidx.at[i]], rows, sem).wait()
            pltpu.sync_copy(rows, agg.at[didx.at[i]], add=True)
            return carry

        lax.fori_loop(0, n_chunks, body, 0)

        plsc.subcore_barrier()

        pltpu.sync_copy(agg.at[pl.ds(row0, rows_per_tile)],
                        out_hbm.at[cid, pl.ds(row0, rows_per_tile)])

    return k(x, src3, dst3)


def _tc_finish(partials, x, W_rel, W_root, b2, g2, be2):
    CH = 1000
    n_ch = N_NODES // CH

    def body(p_ref, x_ref, wr_ref, wt_ref, b_ref, g_ref, be_ref, o_ref):
        wr = wr_ref[...]
        wt = wt_ref[...]
        b = b_ref[...]

        def h_chunk(i):
            a = p_ref[0, pl.ds(i * CH, CH), :] + p_ref[1, pl.ds(i * CH, CH), :]
            xs = x_ref[pl.ds(i * CH, CH), :]
            h = lax.dot_general(a, wr, (((1,), (1,)), ((), ())),
                                preferred_element_type=jnp.float32)
            h = h + lax.dot_general(xs, wt, (((1,), (1,)), ((), ())),
                                    preferred_element_type=jnp.float32)
            return h + b

        def stats(i, carry):
            s, s2 = carry
            h = h_chunk(i)
            return (s + jnp.sum(h, axis=0, keepdims=True),
                    s2 + jnp.sum(h * h, axis=0, keepdims=True))

        s, s2 = lax.fori_loop(
            0, n_ch, stats,
            (jnp.zeros((1, D), jnp.float32), jnp.zeros((1, D), jnp.float32)))
        mean = s * (1.0 / N_NODES)
        var = s2 * (1.0 / N_NODES) - mean * mean
        scale = g_ref[...] * lax.rsqrt(var + EPS)
        shift = be_ref[...] - mean * scale

        def norm(i, carry):
            h = h_chunk(i)
            o_ref[pl.ds(i * CH, CH), :] = jnp.maximum(h * scale + shift, 0.0)
            return carry

        lax.fori_loop(0, n_ch, norm, 0)

    return pl.pallas_call(
        body,
        out_shape=jax.ShapeDtypeStruct((N_NODES, D), jnp.float32),
    )(partials, x, W_rel, W_root, b2, g2, be2)


def kernel(x, edge_index, batch, W_rel, W_root, b_rel, gamma, beta):
    del batch
    ei = edge_index.astype(jnp.int32)
    E = ei.shape[1]
    per_worker = E // NW
    n_chunks = per_worker // E_CHUNK
    src3 = ei[0].reshape(NW, n_chunks, E_CHUNK)
    dst3 = ei[1].reshape(NW, n_chunks, E_CHUNK)
    partials = _sc_segment_sum(x, src3, dst3, n_chunks)
    return _tc_finish(partials, x, W_rel, W_root,
                      b_rel.reshape(1, D), gamma.reshape(1, D),
                      beta.reshape(1, D))


# speedup vs baseline: 7.5616x; 7.5616x over previous
import functools

import jax
import jax.numpy as jnp
from jax import lax
from jax.experimental import pallas as pl
from jax.experimental.pallas import tpu as pltpu
from jax.experimental.pallas import tpu_sc as plsc

N_NODES = 10000
N_PAD = 10240
D = 128
EPS = 1e-5

NC = 2
NS = 16
NW = NC * NS

E_CHUNK = 80


def _sc_segment_sum(x, src3, dst3, n_chunks):
    rows_per_tile = N_PAD // NS

    mesh = plsc.VectorSubcoreMesh(core_axis_name="c", subcore_axis_name="s")

    @functools.partial(
        pl.kernel,
        out_type=jax.ShapeDtypeStruct((NC, N_PAD, D), jnp.float32),
        mesh=mesh,
        scratch_types=[
            pltpu.VMEM((n_chunks, E_CHUNK), jnp.int32),
            pltpu.VMEM((n_chunks, E_CHUNK), jnp.int32),
            pltpu.VMEM((E_CHUNK, D), jnp.float32),
            pltpu.VMEM_SHARED((N_PAD, D), jnp.float32),
            pltpu.SemaphoreType.DMA,
        ],
    )
    def k(x_hbm, src_hbm, dst_hbm, out_hbm, sidx, didx, rows, agg, sem):
        cid = lax.axis_index("c")
        sid = lax.axis_index("s")
        wid = cid * NS + sid

        zero16 = jnp.zeros((16,), jnp.float32)

        def zfill(r, carry):
            for j in range(D // 16):
                rows[r, pl.ds(j * 16, 16)] = zero16
            return carry

        lax.fori_loop(0, E_CHUNK, zfill, 0)
        row0 = sid * rows_per_tile
        for j in range(rows_per_tile // E_CHUNK):
            pltpu.sync_copy(rows, agg.at[pl.ds(row0 + j * E_CHUNK, E_CHUNK)])

        pltpu.sync_copy(src_hbm.at[wid], sidx)
        pltpu.sync_copy(dst_hbm.at[wid], didx)

        plsc.subcore_barrier()

        def body(i, carry):
            pltpu.async_copy(x_hbm.at[sidx.at[i]], rows, sem).wait()
            pltpu.sync_copy(rows, agg.at[didx.at[i]], add=True)
            return carry

        lax.fori_loop(0, n_chunks, body, 0)

        plsc.subcore_barrier()

        pltpu.sync_copy(agg.at[pl.ds(row0, rows_per_tile)],
                        out_hbm.at[cid, pl.ds(row0, rows_per_tile)])

    return k(x, src3, dst3)


def _tc_finish(partials, x, W_rel, W_root, b2, g2, be2):
    CH = 1000
    n_ch = N_NODES // CH

    def body(p_ref, x_ref, wr_ref, wt_ref, b_ref, g_ref, be_ref, o_ref):
        wr = wr_ref[...]
        wt = wt_ref[...]
        b = b_ref[...]

        def h_chunk(i):
            a = p_ref[0, pl.ds(i * CH, CH), :] + p_ref[1, pl.ds(i * CH, CH), :]
            xs = x_ref[pl.ds(i * CH, CH), :]
            h = lax.dot_general(a, wr, (((1,), (1,)), ((), ())),
                                preferred_element_type=jnp.float32)
            h = h + lax.dot_general(xs, wt, (((1,), (1,)), ((), ())),
                                    preferred_element_type=jnp.float32)
            return h + b

        def stats(i, carry):
            s, s2 = carry
            h = h_chunk(i)
            return (s + jnp.sum(h, axis=0, keepdims=True),
                    s2 + jnp.sum(h * h, axis=0, keepdims=True))

        s, s2 = lax.fori_loop(
            0, n_ch, stats,
            (jnp.zeros((1, D), jnp.float32), jnp.zeros((1, D), jnp.float32)))
        mean = s * (1.0 / N_NODES)
        var = s2 * (1.0 / N_NODES) - mean * mean
        scale = g_ref[...] * lax.rsqrt(var + EPS)
        shift = be_ref[...] - mean * scale

        def norm(i, carry):
            h = h_chunk(i)
            o_ref[pl.ds(i * CH, CH), :] = jnp.maximum(h * scale + shift, 0.0)
            return carry

        lax.fori_loop(0, n_ch, norm, 0)

    return pl.pallas_call(
        body,
        out_shape=jax.ShapeDtypeStruct((N_NODES, D), jnp.float32),
    )(partials, x, W_rel, W_root, b2, g2, be2)


def kernel(x, edge_index, batch, W_rel, W_root, b_rel, gamma, beta):
    del batch
    ei = edge_index.astype(jnp.int32)
    E = ei.shape[1]
    per_worker = E // NW
    n_chunks = per_worker // E_CHUNK
    src3 = ei[0].reshape(NW, n_chunks, E_CHUNK)
    dst3 = ei[1].reshape(NW, n_chunks, E_CHUNK)
    partials = _sc_segment_sum(x, src3, dst3, n_chunks)
    return _tc_finish(partials, x, W_rel, W_root,
                      b_rel.reshape(1, D), gamma.reshape(1, D),
                      beta.reshape(1, D))


# speedup vs baseline: 11.5502x; 1.5275x over previous
import functools

import jax
import jax.numpy as jnp
from jax import lax
from jax.experimental import pallas as pl
from jax.experimental.pallas import tpu as pltpu
from jax.experimental.pallas import tpu_sc as plsc

N_NODES = 10000
N_PAD = 10240
D = 128
EPS = 1e-5

NC = 2
NS = 16
NW = NC * NS

E_CHUNK = 80


def _sc_segment_sum(x, src3, dst3, n_chunks):
    rows_per_tile = N_PAD // NS

    mesh = plsc.VectorSubcoreMesh(core_axis_name="c", subcore_axis_name="s")

    @functools.partial(
        pl.kernel,
        out_type=jax.ShapeDtypeStruct((NC, N_PAD, D), jnp.float32),
        mesh=mesh,
        scratch_types=[
            pltpu.VMEM((n_chunks, E_CHUNK), jnp.int32),
            pltpu.VMEM((E_CHUNK,), jnp.int32),
            pltpu.VMEM((E_CHUNK,), jnp.int32),
            pltpu.VMEM((E_CHUNK, D), jnp.float32),
            pltpu.VMEM((E_CHUNK, D), jnp.float32),
            pltpu.VMEM_SHARED((N_PAD, D), jnp.float32),
            pltpu.SemaphoreType.DMA,
            pltpu.SemaphoreType.DMA,
            pltpu.SemaphoreType.DMA,
            pltpu.SemaphoreType.DMA,
        ],
    )
    def k(x_hbm, src_hbm, dst_hbm, out_hbm, sidx, didx_a, didx_b,
          rows_a, rows_b, agg, sem_a, sem_b, dsem_a, dsem_b):
        cid = lax.axis_index("c")
        sid = lax.axis_index("s")
        wid = cid * NS + sid

        zero16 = jnp.zeros((16,), jnp.float32)

        def zfill(r, carry):
            for j in range(D // 16):
                rows_a[r, pl.ds(j * 16, 16)] = zero16
            return carry

        lax.fori_loop(0, E_CHUNK, zfill, 0)
        row0 = sid * rows_per_tile
        for j in range(rows_per_tile // E_CHUNK):
            pltpu.sync_copy(rows_a, agg.at[pl.ds(row0 + j * E_CHUNK, E_CHUNK)])

        pltpu.sync_copy(src_hbm.at[wid], sidx)

        plsc.subcore_barrier()

        pltpu.async_copy(dst_hbm.at[wid, 0], didx_a, dsem_a)
        pltpu.async_copy(x_hbm.at[sidx.at[0]], rows_a, sem_a)

        def step(i, rows, sem, didx, dsem, nrows, nsem, ndidx, ndsem):
            @pl.when(i + 1 < n_chunks)
            def _():
                pltpu.async_copy(dst_hbm.at[wid, i + 1], ndidx, ndsem)
                pltpu.async_copy(x_hbm.at[sidx.at[i + 1]], nrows, nsem)

            pltpu.make_async_copy(x_hbm.at[sidx.at[i]], rows, sem).wait()
            pltpu.make_async_copy(dst_hbm.at[wid, i], didx, dsem).wait()
            pltpu.sync_copy(rows, agg.at[didx], add=True)

        def body(i, carry):
            @pl.when(lax.rem(i, 2) == 0)
            def _():
                step(i, rows_a, sem_a, didx_a, dsem_a,
                     rows_b, sem_b, didx_b, dsem_b)

            @pl.when(lax.rem(i, 2) == 1)
            def _():
                step(i, rows_b, sem_b, didx_b, dsem_b,
                     rows_a, sem_a, didx_a, dsem_a)

            return carry

        lax.fori_loop(0, n_chunks, body, 0)

        plsc.subcore_barrier()

        pltpu.sync_copy(agg.at[pl.ds(row0, rows_per_tile)],
                        out_hbm.at[cid, pl.ds(row0, rows_per_tile)])

    return k(x, src3, dst3)


def _tc_finish(partials, x, W_rel, W_root, b2, g2, be2):
    CH = 1000
    n_ch = N_NODES // CH

    def body(p_ref, x_ref, wr_ref, wt_ref, b_ref, g_ref, be_ref, o_ref):
        wr = wr_ref[...]
        wt = wt_ref[...]
        b = b_ref[...]

        def h_chunk(i):
            a = p_ref[0, pl.ds(i * CH, CH), :] + p_ref[1, pl.ds(i * CH, CH), :]
            xs = x_ref[pl.ds(i * CH, CH), :]
            h = lax.dot_general(a, wr, (((1,), (1,)), ((), ())),
                                preferred_element_type=jnp.float32)
            h = h + lax.dot_general(xs, wt, (((1,), (1,)), ((), ())),
                                    preferred_element_type=jnp.float32)
            return h + b

        def stats(i, carry):
            s, s2 = carry
            h = h_chunk(i)
            return (s + jnp.sum(h, axis=0, keepdims=True),
                    s2 + jnp.sum(h * h, axis=0, keepdims=True))

        s, s2 = lax.fori_loop(
            0, n_ch, stats,
            (jnp.zeros((1, D), jnp.float32), jnp.zeros((1, D), jnp.float32)))
        mean = s * (1.0 / N_NODES)
        var = s2 * (1.0 / N_NODES) - mean * mean
        scale = g_ref[...] * lax.rsqrt(var + EPS)
        shift = be_ref[...] - mean * scale

        def norm(i, carry):
            h = h_chunk(i)
            o_ref[pl.ds(i * CH, CH), :] = jnp.maximum(h * scale + shift, 0.0)
            return carry

        lax.fori_loop(0, n_ch, norm, 0)

    return pl.pallas_call(
        body,
        out_shape=jax.ShapeDtypeStruct((N_NODES, D), jnp.float32),
    )(partials, x, W_rel, W_root, b2, g2, be2)


def kernel(x, edge_index, batch, W_rel, W_root, b_rel, gamma, beta):
    del batch
    ei = edge_index.astype(jnp.int32)
    E = ei.shape[1]
    per_worker = E // NW
    n_chunks = per_worker // E_CHUNK
    src3 = ei[0].reshape(NW, n_chunks, E_CHUNK)
    dst3 = ei[1].reshape(NW, n_chunks, E_CHUNK)
    partials = _sc_segment_sum(x, src3, dst3, n_chunks)
    return _tc_finish(partials, x, W_rel, W_root,
                      b_rel.reshape(1, D), gamma.reshape(1, D),
                      beta.reshape(1, D))
